# baseline (speedup 1.0000x reference)
import functools

import jax
import jax.numpy as jnp
from jax import lax
from jax.experimental import pallas as pl
from jax.experimental.pallas import tpu as pltpu
from jax.experimental.pallas import tpu_sc as plsc

DIM = 128
TIMESTEPS = 1000
BATCH = 16384

NUM_CORES = 2
NUM_SUBCORES = 16
NW = NUM_CORES * NUM_SUBCORES
B_PER_W = BATCH // NW
CHUNK = 128
N_CHUNKS = B_PER_W // CHUNK


@functools.partial(
    pl.kernel,
    mesh=plsc.VectorSubcoreMesh(core_axis_name="c", subcore_axis_name="s"),
    out_type=jax.ShapeDtypeStruct((BATCH, DIM), jnp.float32),
    scratch_types=[
        pltpu.VMEM((N_CHUNKS, CHUNK), jnp.int32),
        pltpu.VMEM((N_CHUNKS, CHUNK, DIM), jnp.float32),
        pltpu.SemaphoreType.DMA((N_CHUNKS,)),
        pltpu.SemaphoreType.DMA,
    ],
)
def _gather_kernel(idx_hbm, table_hbm, out_hbm, idx_v, rows_v, gsem, wsem):
    wid = lax.axis_index("s") * NUM_CORES + lax.axis_index("c")
    pltpu.sync_copy(idx_hbm.at[wid], idx_v)
    gathers = [
        pltpu.async_copy(table_hbm.at[idx_v.at[j]], rows_v.at[j], gsem.at[j])
        for j in range(N_CHUNKS)
    ]
    base = wid * B_PER_W
    writes = []
    for j in range(N_CHUNKS):
        gathers[j].wait()
        writes.append(
            pltpu.async_copy(
                rows_v.at[j], out_hbm.at[pl.ds(base + j * CHUNK, CHUNK)], wsem
            )
        )
    for w in writes:
        w.wait()


def kernel(timestep, pe_matrix):
    idx = timestep.astype(jnp.int32).reshape(NW, N_CHUNKS, CHUNK)
    return _gather_kernel(idx, pe_matrix)


# speedup vs baseline: 1.2398x; 1.2398x over previous
import functools

import jax
import jax.numpy as jnp
from jax import lax
from jax.experimental import pallas as pl
from jax.experimental.pallas import tpu as pltpu
from jax.experimental.pallas import tpu_sc as plsc

DIM = 128
TIMESTEPS = 1000
BATCH = 16384

NUM_CORES = 2
NUM_SUBCORES = 16
NW = NUM_CORES * NUM_SUBCORES
B_PER_W = BATCH // NW
CHUNK = 128
N_CHUNKS = B_PER_W // CHUNK
STAGE_ROWS = 64
TAIL_BASE = STAGE_ROWS * (NUM_SUBCORES - 1)
TAIL_ROWS = TIMESTEPS - TAIL_BASE


@functools.partial(
    pl.kernel,
    mesh=plsc.VectorSubcoreMesh(core_axis_name="c", subcore_axis_name="s"),
    out_type=jax.ShapeDtypeStruct((BATCH, DIM), jnp.float32),
    scratch_types=[
        pltpu.VMEM((N_CHUNKS, CHUNK), jnp.int32),
        pltpu.VMEM((N_CHUNKS, CHUNK, DIM), jnp.float32),
        pltpu.VMEM_SHARED((TIMESTEPS, DIM), jnp.float32),
        pltpu.SemaphoreType.DMA((N_CHUNKS,)),
        pltpu.SemaphoreType.DMA,
    ],
)
def _gather_kernel(idx_hbm, table_hbm, out_hbm, idx_v, rows_v, shared, gsem, wsem):
    cid = lax.axis_index("c")
    sid = lax.axis_index("s")
    wid = sid * NUM_CORES + cid
    pltpu.sync_copy(idx_hbm.at[wid], idx_v)
    @pl.when(sid < NUM_SUBCORES - 1)
    def _stage_body():
        pltpu.sync_copy(
            table_hbm.at[pl.ds(sid * STAGE_ROWS, STAGE_ROWS)],
            shared.at[pl.ds(sid * STAGE_ROWS, STAGE_ROWS)],
        )

    @pl.when(sid == NUM_SUBCORES - 1)
    def _stage_tail():
        pltpu.sync_copy(
            table_hbm.at[pl.ds(TAIL_BASE, TAIL_ROWS)],
            shared.at[pl.ds(TAIL_BASE, TAIL_ROWS)],
        )

    plsc.subcore_barrier()
    gathers = [
        pltpu.async_copy(shared.at[idx_v.at[j]], rows_v.at[j], gsem.at[j])
        for j in range(N_CHUNKS)
    ]
    base = wid * B_PER_W
    writes = []
    for j in range(N_CHUNKS):
        gathers[j].wait()
        writes.append(
            pltpu.async_copy(
                rows_v.at[j], out_hbm.at[pl.ds(base + j * CHUNK, CHUNK)], wsem
            )
        )
    for w in writes:
        w.wait()


def kernel(timestep, pe_matrix):
    idx = timestep.astype(jnp.int32).reshape(NW, N_CHUNKS, CHUNK)
    return _gather_kernel(idx, pe_matrix)
